# baseline (speedup 1.0000x reference)
import functools

import jax
import jax.numpy as jnp
import numpy as np
from jax import lax
from jax.experimental import pallas as pl
from jax.experimental.pallas import tpu as pltpu
from jax.experimental.pallas import tpu_sc as plsc

_ALPHA = 0.5
_NC = 2
_NS = 16
_PACKED = False
_CHUNK = 64 if _PACKED else 128


def _ceil_to(x, m):
  return (x + m - 1) // m * m


def _prologue_body(x_ref, od_ref, id_ref, y_ref, z_ref):
  x = x_ref[...]
  od = od_ref[...].astype(jnp.float32)
  idg = id_ref[...].astype(jnp.float32)
  inv_o = jnp.where(od > 0, lax.rsqrt(od), 0.0)
  inv_i = jnp.where(idg > 0, lax.rsqrt(idg), 0.0)
  y_ref[...] = (x * inv_i).astype(_TBL_DTYPE)
  z_ref[...] = (x * inv_o).astype(_TBL_DTYPE)


def _prologue(x_pad, od_pad, id_pad, n_pad, d):
  blk = 1024
  grid = (n_pad // blk,)
  return pl.pallas_call(
      _prologue_body,
      grid=grid,
      in_specs=[
          pl.BlockSpec((blk, d), lambda i: (i, 0)),
          pl.BlockSpec((blk, 1), lambda i: (i, 0)),
          pl.BlockSpec((blk, 1), lambda i: (i, 0)),
      ],
      out_specs=[
          pl.BlockSpec((blk, d), lambda i: (i, 0)),
          pl.BlockSpec((blk, d), lambda i: (i, 0)),
      ],
      out_shape=[
          jax.ShapeDtypeStruct((n_pad, d), _TBL_DTYPE),
          jax.ShapeDtypeStruct((n_pad, d), _TBL_DTYPE),
      ],
  )(x_pad, od_pad, id_pad)


_NBUF = 4 if _PACKED else 1
_NFB = 2
_SEG = 32
_TBL_DTYPE = jnp.bfloat16 if _PACKED else jnp.float32
_GATHER_D = 64


def _sc_aggregate(y, z, row_p, col_p, odeg_p, ideg_p, out_tab, in_tab,
                  zeros_tile, n_pad, d, e_pad):
  epw = e_pad // _NS
  n_echunks = epw // _CHUNK
  n_segs = n_echunks // _SEG
  rpt = n_pad // _NS
  n_rchunks = rpt // _CHUNK

  mesh = plsc.VectorSubcoreMesh(core_axis_name="c", subcore_axis_name="s",
                                num_cores=_NC, num_subcores=_NS)

  @functools.partial(
      pl.kernel,
      out_type=[jax.ShapeDtypeStruct((n_pad, d), jnp.float32)] * 4,
      mesh=mesh,
      scratch_types=[
          pltpu.VMEM((_SEG, _CHUNK), jnp.int32),
          pltpu.VMEM((_SEG, _CHUNK), jnp.int32),
          (pltpu.VMEM((_NBUF, _CHUNK, _GATHER_D), jnp.int32) if _PACKED
           else pltpu.VMEM((_NBUF, _CHUNK, d), jnp.float32)),
          (pltpu.VMEM((_NFB, _CHUNK, d), jnp.float32) if _PACKED
           else pltpu.VMEM((1, 16), jnp.float32)),
          pltpu.VMEM((_CHUNK, d), jnp.float32),
          pltpu.VMEM_SHARED((n_pad, d), jnp.float32),
          [pltpu.SemaphoreType.DMA] * _NBUF,
          [pltpu.SemaphoreType.DMA] * _NFB,
          pltpu.SemaphoreType.DMA,
      ],
      compiler_params=pltpu.CompilerParams(use_tc_tiling_on_sc=False,
                                           needs_layout_passes=False),
  )
  def sc_kernel(y_hbm, z_hbm, row_hbm, col_hbm, odeg_hbm, ideg_hbm,
                otab_hbm, itab_hbm, zeros_hbm,
                oacc_hbm, iacc_hbm, otabg_hbm, itabg_hbm,
                gidx_v, sidx_v, rows_v, rowsf_v, tabrow_v, acc_sh,
                gsems, ssems, sem):
    c = lax.axis_index("c")
    s = lax.axis_index("s")

    pltpu.sync_copy(zeros_hbm, acc_sh.at[pl.ds(s * rpt, rpt)])
    plsc.subcore_barrier()

    def run_edges(tbl_hbm, g_hbm, s_hbm):
      cbase = s * n_echunks

      def gather_desc(j, b):
        return pltpu.make_async_copy(tbl_hbm.at[gidx_v.at[j]], rows_v.at[b],
                                     gsems[b])

      def scatter_desc(j, fb):
        return pltpu.make_async_copy(rowsf_v.at[fb], acc_sh.at[sidx_v.at[j]],
                                     ssems[fb])

      def unpack_chunk(b, fb):
        def rbody(r, carry3):
          for cq in range(_GATHER_D // 16):
            w = rows_v[b, r, pl.ds(cq * 16, 16)]
            bf = plsc.bitcast(w, jnp.bfloat16)
            lo, hi = plsc.unpack(bf, format=plsc.PackFormat.INTERLEAVED)
            rowsf_v[fb, r, pl.ds(cq * 32, 16)] = lo
            rowsf_v[fb, r, pl.ds(cq * 32 + 16, 16)] = hi
          return carry3

        lax.fori_loop(0, _CHUNK, rbody, 0)

      def segment_sync(seg, carry):
        sb = cbase + seg * _SEG
        pltpu.sync_copy(g_hbm.at[pl.ds(sb, _SEG)], gidx_v)
        pltpu.sync_copy(s_hbm.at[pl.ds(sb, _SEG)], sidx_v)

        def chunk(j, carry2):
          pltpu.sync_copy(tbl_hbm.at[gidx_v.at[j]], rows_v.at[0])
          pltpu.sync_copy(rows_v.at[0], acc_sh.at[sidx_v.at[j]], add=True)
          return carry2

        lax.fori_loop(0, _SEG, chunk, 0)
        return carry

      def segment(seg, carry):
        sb = cbase + seg * _SEG
        pltpu.sync_copy(g_hbm.at[pl.ds(sb, _SEG)], gidx_v)
        pltpu.sync_copy(s_hbm.at[pl.ds(sb, _SEG)], sidx_v)

        for b0 in range(_NBUF - 1):
          gather_desc(b0, b0).start()

        def group(gr, carry2):
          for b in range(_NBUF):
            j = gr * _NBUF + b
            pb = (b - 1) % _NBUF
            fb = b % _NFB
            gather_desc(j, b).wait()

            @pl.when(j >= _NFB)
            def _():
              scatter_desc(j - _NFB, fb).wait()

            unpack_chunk(b, fb)
            scatter_desc(j, fb).start(add=True)

            @pl.when(j + _NBUF - 1 < _SEG)
            def _():
              gather_desc(j + _NBUF - 1, pb).start()
          return carry2

        lax.fori_loop(0, _SEG // _NBUF, group, 0)
        for k in range(_NFB):
          jj = _SEG - _NFB + k
          scatter_desc(jj, jj % _NFB).wait()
        return carry

      lax.fori_loop(0, n_segs, segment if _PACKED else segment_sync, 0)

    def run_tab(tab_hbm, deg_hbm, tabg_hbm):
      rbase = s * rpt

      def body(j, carry):
        rb = rbase + j * _CHUNK
        pltpu.sync_copy(deg_hbm.at[pl.ds(rb, _CHUNK)], gidx_v.at[0])
        pltpu.async_copy(tab_hbm.at[gidx_v.at[0]], tabrow_v, sem).wait()
        pltpu.sync_copy(tabrow_v, tabg_hbm.at[pl.ds(rb, _CHUNK)])
        return carry

      lax.fori_loop(0, n_rchunks, body, 0)

    @pl.when(c == 0)
    def _():
      run_edges(y_hbm, col_hbm, row_hbm)

    @pl.when(c == 1)
    def _():
      run_edges(z_hbm, row_hbm, col_hbm)

    plsc.subcore_barrier()

    @pl.when(c == 0)
    def _():
      pltpu.sync_copy(acc_sh.at[pl.ds(s * rpt, rpt)],
                      oacc_hbm.at[pl.ds(s * rpt, rpt)])
      run_tab(otab_hbm, odeg_hbm, otabg_hbm)

    @pl.when(c == 1)
    def _():
      pltpu.sync_copy(acc_sh.at[pl.ds(s * rpt, rpt)],
                      iacc_hbm.at[pl.ds(s * rpt, rpt)])
      run_tab(itab_hbm, ideg_hbm, itabg_hbm)

  return sc_kernel(y, z, row_p, col_p, odeg_p, ideg_p, out_tab, in_tab,
                   zeros_tile)


def _epilogue_body(x_ref, oacc_ref, iacc_ref, otg_ref, itg_ref,
                   od_ref, id_ref, om_ref, omb_ref, im_ref, imb_ref,
                   wsd_ref, bsd_ref, wds_ref, bds_ref,
                   wof_ref, wofp_ref, bof_ref, wif_ref, wifp_ref, bif_ref,
                   wfc_ref, bfc_ref,
                   out_ref, co_ref, ci_ref):
  x = x_ref[...]
  od = od_ref[...].astype(jnp.float32)
  idg = id_ref[...].astype(jnp.float32)
  inv_o = jnp.where(od > 0, lax.rsqrt(od), 0.0)
  inv_i = jnp.where(idg > 0, lax.rsqrt(idg), 0.0)
  out_nei = inv_o * oacc_ref[...]
  in_nei = inv_i * iacc_ref[...]

  co_s = (jnp.sum(out_nei * wofp_ref[...], axis=1, keepdims=True)
          + jnp.sum((otg_ref[...] - x) * wof_ref[...], axis=1, keepdims=True)
          + bof_ref[...])
  ci_s = (jnp.sum(in_nei * wifp_ref[...], axis=1, keepdims=True)
          + jnp.sum((itg_ref[...] - x) * wif_ref[...], axis=1, keepdims=True)
          + bif_ref[...])
  m = jnp.maximum(co_s, ci_s)
  eo = jnp.exp(co_s - m)
  ei = jnp.exp(ci_s - m)
  denom = eo + ei
  c_out = (eo / denom) * om_ref[...] + omb_ref[...]
  c_in = (ei / denom) * im_ref[...] + imb_ref[...]

  acc = jnp.dot(x, wfc_ref[...], preferred_element_type=jnp.float32)
  acc = _ALPHA * (acc + bfc_ref[...])
  acc = acc + c_out * (
      jnp.dot(out_nei, wsd_ref[...], preferred_element_type=jnp.float32)
      + bsd_ref[...])
  acc = acc + c_in * (
      jnp.dot(in_nei, wds_ref[...], preferred_element_type=jnp.float32)
      + bds_ref[...])
  out_ref[...] = acc
  co_ref[...] = c_out
  ci_ref[...] = c_in


def _epilogue(x_pad, oacc, iacc, otg, itg, od_pad, id_pad,
              om, omb, im, imb,
              W_sd, b_sd, W_ds, b_ds, wof_t, wofp_t, bof, wif_t, wifp_t, bif,
              W_fc, b_fc, n_pad, d, out_dim):
  blk = 512
  grid = (n_pad // blk,)
  row_spec = pl.BlockSpec((blk, d), lambda i: (i, 0))
  col1_spec = pl.BlockSpec((blk, 1), lambda i: (i, 0))
  w_spec = pl.BlockSpec((d, out_dim), lambda i: (0, 0))
  b_spec = pl.BlockSpec((1, out_dim), lambda i: (0, 0))
  vrow_spec = pl.BlockSpec((1, d), lambda i: (0, 0))
  s_spec = pl.BlockSpec((1, 1), lambda i: (0, 0))
  return pl.pallas_call(
      _epilogue_body,
      grid=grid,
      in_specs=[
          row_spec, row_spec, row_spec, row_spec, row_spec,
          col1_spec, col1_spec, col1_spec, col1_spec, col1_spec, col1_spec,
          w_spec, b_spec, w_spec, b_spec,
          vrow_spec, vrow_spec, s_spec, vrow_spec, vrow_spec, s_spec,
          w_spec, b_spec,
      ],
      out_specs=[
          pl.BlockSpec((blk, out_dim), lambda i: (i, 0)),
          col1_spec,
          col1_spec,
      ],
      out_shape=[
          jax.ShapeDtypeStruct((n_pad, out_dim), jnp.float32),
          jax.ShapeDtypeStruct((n_pad, 1), jnp.float32),
          jax.ShapeDtypeStruct((n_pad, 1), jnp.float32),
      ],
  )(x_pad, oacc, iacc, otg, itg, od_pad, id_pad, om, omb, im, imb,
    W_sd, b_sd, W_ds, b_ds, wof_t, wofp_t, bof, wif_t, wifp_t, bif,
    W_fc, b_fc)


@jax.jit
def _run(x, edge_index, in_degree, out_degree, in_tab, out_tab,
         W_sd, b_sd, W_ds, b_ds, w_out_f, b_out_f, w_in_f, b_in_f,
         W_fc, b_fc, out_deg_mask, out_deg_mask_bias,
         in_deg_mask, in_deg_mask_bias):
  n, d = x.shape
  e = edge_index.shape[1]
  out_dim = W_sd.shape[1]

  n_pad = _ceil_to(n, _NS * _CHUNK)
  e_pad = _ceil_to(e, _NS * _CHUNK * _SEG)

  x_pad = jnp.pad(x, ((0, n_pad - n), (0, 0)))
  od_pad = jnp.pad(out_degree, (0, n_pad - n)).reshape(n_pad, 1)
  id_pad = jnp.pad(in_degree, (0, n_pad - n)).reshape(n_pad, 1)
  row_p = jnp.pad(edge_index[0], (0, e_pad - e),
                  constant_values=n_pad - 1).reshape(e_pad // _CHUNK, _CHUNK)
  col_p = jnp.pad(edge_index[1], (0, e_pad - e),
                  constant_values=0).reshape(e_pad // _CHUNK, _CHUNK)

  y, z = _prologue(x_pad, od_pad, id_pad, n_pad, d)
  if _PACKED:
    y = jax.lax.bitcast_convert_type(y.reshape(n_pad, d // 2, 2), jnp.int32)
    z = jax.lax.bitcast_convert_type(z.reshape(n_pad, d // 2, 2), jnp.int32)

  zeros_tile = jnp.zeros((n_pad // _NS, d), jnp.float32)
  odeg_flat = od_pad.reshape(n_pad)
  ideg_flat = id_pad.reshape(n_pad)
  oacc, iacc, otg, itg = _sc_aggregate(
      y, z, row_p, col_p, odeg_flat, ideg_flat, out_tab, in_tab,
      zeros_tile, n_pad, d, e_pad)

  if _PACKED:
    sigma = np.empty(d, dtype=np.int32)
    for g in range(d // 32):
      for i in range(16):
        sigma[32 * g + i] = 32 * g + 2 * i
        sigma[32 * g + 16 + i] = 32 * g + 2 * i + 1
  else:
    sigma = np.arange(d, dtype=np.int32)

  pad1 = lambda v: jnp.pad(v, (0, n_pad - n)).reshape(n_pad, 1)
  out, co, ci = _epilogue(
      x_pad, oacc, iacc, otg, itg, od_pad, id_pad,
      pad1(out_deg_mask), pad1(out_deg_mask_bias),
      pad1(in_deg_mask), pad1(in_deg_mask_bias),
      W_sd[sigma, :], b_sd.reshape(1, out_dim),
      W_ds[sigma, :], b_ds.reshape(1, out_dim),
      w_out_f.reshape(1, d), w_out_f[sigma, :].reshape(1, d),
      b_out_f.reshape(1, 1),
      w_in_f.reshape(1, d), w_in_f[sigma, :].reshape(1, d),
      b_in_f.reshape(1, 1),
      W_fc, b_fc.reshape(1, out_dim),
      n_pad, d, out_dim)

  return out[:n], ci[:n], co[:n]


def kernel(x, edge_index, in_degree, out_degree, in_tab, out_tab,
           W_sd, b_sd, W_ds, b_ds, w_out_f, b_out_f, w_in_f, b_in_f,
           W_fc, b_fc, out_deg_mask, out_deg_mask_bias,
           in_deg_mask, in_deg_mask_bias):
  return _run(x, edge_index, in_degree, out_degree, in_tab, out_tab,
              W_sd, b_sd, W_ds, b_ds, w_out_f, b_out_f, w_in_f, b_in_f,
              W_fc, b_fc, out_deg_mask, out_deg_mask_bias,
              in_deg_mask, in_deg_mask_bias)


# speedup vs baseline: 1.4680x; 1.4680x over previous
import functools

import jax
import jax.numpy as jnp
import numpy as np
from jax import lax
from jax.experimental import pallas as pl
from jax.experimental.pallas import tpu as pltpu
from jax.experimental.pallas import tpu_sc as plsc

_ALPHA = 0.5
_NC = 2
_NS = 16
_PACKED = False
_CHUNK = 64 if _PACKED else 128


def _ceil_to(x, m):
  return (x + m - 1) // m * m


def _prologue_body(x_ref, od_ref, id_ref, y_ref, z_ref):
  x = x_ref[...]
  od = od_ref[...].astype(jnp.float32)
  idg = id_ref[...].astype(jnp.float32)
  inv_o = jnp.where(od > 0, lax.rsqrt(od), 0.0)
  inv_i = jnp.where(idg > 0, lax.rsqrt(idg), 0.0)
  y_ref[...] = (x * inv_i).astype(_TBL_DTYPE)
  z_ref[...] = (x * inv_o).astype(_TBL_DTYPE)


def _prologue(x_pad, od_pad, id_pad, n_pad, d):
  blk = 1024
  grid = (n_pad // blk,)
  return pl.pallas_call(
      _prologue_body,
      grid=grid,
      in_specs=[
          pl.BlockSpec((blk, d), lambda i: (i, 0)),
          pl.BlockSpec((blk, 1), lambda i: (i, 0)),
          pl.BlockSpec((blk, 1), lambda i: (i, 0)),
      ],
      out_specs=[
          pl.BlockSpec((blk, d), lambda i: (i, 0)),
          pl.BlockSpec((blk, d), lambda i: (i, 0)),
      ],
      out_shape=[
          jax.ShapeDtypeStruct((n_pad, d), _TBL_DTYPE),
          jax.ShapeDtypeStruct((n_pad, d), _TBL_DTYPE),
      ],
  )(x_pad, od_pad, id_pad)


_TBL_DTYPE = jnp.bfloat16 if _PACKED else jnp.float32


def _sc_aggregate(y, z, row_p, col_p, odeg_p, ideg_p, out_tab, in_tab,
                  zeros_tile, n_pad, d, e_pad):
  epw = e_pad // _NS
  n_echunks = epw // _CHUNK
  n_groups = n_echunks // 2
  rpt = n_pad // _NS
  n_rchunks = rpt // _CHUNK

  mesh = plsc.VectorSubcoreMesh(core_axis_name="c", subcore_axis_name="s",
                                num_cores=_NC, num_subcores=_NS)

  @functools.partial(
      pl.kernel,
      out_type=[jax.ShapeDtypeStruct((n_pad, d), jnp.float32)] * 4,
      mesh=mesh,
      scratch_types=[
          [pltpu.VMEM((_CHUNK,), jnp.int32)] * 2,
          [pltpu.VMEM((_CHUNK,), jnp.int32)] * 2,
          [pltpu.VMEM((_CHUNK, d), jnp.float32)] * 2,
          [pltpu.SemaphoreType.DMA] * 2,
          [pltpu.SemaphoreType.DMA] * 2,
          [pltpu.SemaphoreType.DMA] * 2,
          pltpu.VMEM_SHARED((n_pad, d), jnp.float32),
          pltpu.SemaphoreType.DMA,
      ],
      compiler_params=pltpu.CompilerParams(use_tc_tiling_on_sc=False,
                                           needs_layout_passes=False),
  )
  def sc_kernel(y_hbm, z_hbm, row_hbm, col_hbm, odeg_hbm, ideg_hbm,
                otab_hbm, itab_hbm, zeros_hbm,
                oacc_hbm, iacc_hbm, otabg_hbm, itabg_hbm,
                gidx, sidx, rows, isems, gsems, ssems, acc_sh, sem):
    c = lax.axis_index("c")
    s = lax.axis_index("s")

    pltpu.sync_copy(zeros_hbm, acc_sh.at[pl.ds(s * rpt, rpt)])
    plsc.subcore_barrier()

    def run_edges(tbl_hbm, g_hbm, s_hbm):
      tbase = s * epw

      def idx_descs(i, p):
        eb = tbase + i * _CHUNK
        return (pltpu.make_async_copy(g_hbm.at[pl.ds(eb, _CHUNK)], gidx[p],
                                      isems[p]),
                pltpu.make_async_copy(s_hbm.at[pl.ds(eb, _CHUNK)], sidx[p],
                                      isems[p]))

      def gather_desc(p):
        return pltpu.make_async_copy(tbl_hbm.at[gidx[p]], rows[p], gsems[p])

      def scatter_desc(p):
        return pltpu.make_async_copy(rows[p], acc_sh.at[sidx[p]], ssems[p])

      def start_idx(i, p):
        da, db = idx_descs(i, p)
        da.start()
        db.start()

      def wait_idx(i, p):
        da, db = idx_descs(i, p)
        da.wait()
        db.wait()

      start_idx(0, 0)

      def group(g, carry):
        for p in (0, 1):
          i = 2 * g + p
          wait_idx(i, p)
          gather_desc(p).start()

          if p == 0:
            @pl.when(g >= 1)
            def _():
              scatter_desc(1).wait()

            start_idx(i + 1, 1)
          else:
            scatter_desc(0).wait()

            @pl.when(g < n_groups - 1)
            def _():
              start_idx(i + 1, 0)

          gather_desc(p).wait()
          scatter_desc(p).start(add=True)
        return carry

      lax.fori_loop(0, n_groups, group, 0)
      scatter_desc(1).wait()

    def run_tab(tab_hbm, deg_hbm, tabg_hbm):
      rbase = s * rpt

      def body(j, carry):
        rb = rbase + j * _CHUNK
        pltpu.sync_copy(deg_hbm.at[pl.ds(rb, _CHUNK)], gidx[0])
        pltpu.async_copy(tab_hbm.at[gidx[0]], rows[0], sem).wait()
        pltpu.sync_copy(rows[0], tabg_hbm.at[pl.ds(rb, _CHUNK)])
        return carry

      lax.fori_loop(0, n_rchunks, body, 0)

    @pl.when(c == 0)
    def _():
      run_edges(y_hbm, col_hbm, row_hbm)

    @pl.when(c == 1)
    def _():
      run_edges(z_hbm, row_hbm, col_hbm)

    plsc.subcore_barrier()

    @pl.when(c == 0)
    def _():
      pltpu.sync_copy(acc_sh.at[pl.ds(s * rpt, rpt)],
                      oacc_hbm.at[pl.ds(s * rpt, rpt)])
      run_tab(otab_hbm, odeg_hbm, otabg_hbm)

    @pl.when(c == 1)
    def _():
      pltpu.sync_copy(acc_sh.at[pl.ds(s * rpt, rpt)],
                      iacc_hbm.at[pl.ds(s * rpt, rpt)])
      run_tab(itab_hbm, ideg_hbm, itabg_hbm)

  return sc_kernel(y, z, row_p, col_p, odeg_p, ideg_p, out_tab, in_tab,
                   zeros_tile)


def _epilogue_body(x_ref, oacc_ref, iacc_ref, otg_ref, itg_ref,
                   od_ref, id_ref, om_ref, omb_ref, im_ref, imb_ref,
                   wsd_ref, bsd_ref, wds_ref, bds_ref,
                   wof_ref, wofp_ref, bof_ref, wif_ref, wifp_ref, bif_ref,
                   wfc_ref, bfc_ref,
                   out_ref, co_ref, ci_ref):
  x = x_ref[...]
  od = od_ref[...].astype(jnp.float32)
  idg = id_ref[...].astype(jnp.float32)
  inv_o = jnp.where(od > 0, lax.rsqrt(od), 0.0)
  inv_i = jnp.where(idg > 0, lax.rsqrt(idg), 0.0)
  out_nei = inv_o * oacc_ref[...]
  in_nei = inv_i * iacc_ref[...]

  co_s = (jnp.sum(out_nei * wofp_ref[...], axis=1, keepdims=True)
          + jnp.sum((otg_ref[...] - x) * wof_ref[...], axis=1, keepdims=True)
          + bof_ref[...])
  ci_s = (jnp.sum(in_nei * wifp_ref[...], axis=1, keepdims=True)
          + jnp.sum((itg_ref[...] - x) * wif_ref[...], axis=1, keepdims=True)
          + bif_ref[...])
  m = jnp.maximum(co_s, ci_s)
  eo = jnp.exp(co_s - m)
  ei = jnp.exp(ci_s - m)
  denom = eo + ei
  c_out = (eo / denom) * om_ref[...] + omb_ref[...]
  c_in = (ei / denom) * im_ref[...] + imb_ref[...]

  acc = jnp.dot(x, wfc_ref[...], preferred_element_type=jnp.float32)
  acc = _ALPHA * (acc + bfc_ref[...])
  acc = acc + c_out * (
      jnp.dot(out_nei, wsd_ref[...], preferred_element_type=jnp.float32)
      + bsd_ref[...])
  acc = acc + c_in * (
      jnp.dot(in_nei, wds_ref[...], preferred_element_type=jnp.float32)
      + bds_ref[...])
  out_ref[...] = acc
  co_ref[...] = c_out
  ci_ref[...] = c_in


def _epilogue(x_pad, oacc, iacc, otg, itg, od_pad, id_pad,
              om, omb, im, imb,
              W_sd, b_sd, W_ds, b_ds, wof_t, wofp_t, bof, wif_t, wifp_t, bif,
              W_fc, b_fc, n_pad, d, out_dim):
  blk = 512
  grid = (n_pad // blk,)
  row_spec = pl.BlockSpec((blk, d), lambda i: (i, 0))
  col1_spec = pl.BlockSpec((blk, 1), lambda i: (i, 0))
  w_spec = pl.BlockSpec((d, out_dim), lambda i: (0, 0))
  b_spec = pl.BlockSpec((1, out_dim), lambda i: (0, 0))
  vrow_spec = pl.BlockSpec((1, d), lambda i: (0, 0))
  s_spec = pl.BlockSpec((1, 1), lambda i: (0, 0))
  return pl.pallas_call(
      _epilogue_body,
      grid=grid,
      in_specs=[
          row_spec, row_spec, row_spec, row_spec, row_spec,
          col1_spec, col1_spec, col1_spec, col1_spec, col1_spec, col1_spec,
          w_spec, b_spec, w_spec, b_spec,
          vrow_spec, vrow_spec, s_spec, vrow_spec, vrow_spec, s_spec,
          w_spec, b_spec,
      ],
      out_specs=[
          pl.BlockSpec((blk, out_dim), lambda i: (i, 0)),
          col1_spec,
          col1_spec,
      ],
      out_shape=[
          jax.ShapeDtypeStruct((n_pad, out_dim), jnp.float32),
          jax.ShapeDtypeStruct((n_pad, 1), jnp.float32),
          jax.ShapeDtypeStruct((n_pad, 1), jnp.float32),
      ],
  )(x_pad, oacc, iacc, otg, itg, od_pad, id_pad, om, omb, im, imb,
    W_sd, b_sd, W_ds, b_ds, wof_t, wofp_t, bof, wif_t, wifp_t, bif,
    W_fc, b_fc)


@jax.jit
def _run(x, edge_index, in_degree, out_degree, in_tab, out_tab,
         W_sd, b_sd, W_ds, b_ds, w_out_f, b_out_f, w_in_f, b_in_f,
         W_fc, b_fc, out_deg_mask, out_deg_mask_bias,
         in_deg_mask, in_deg_mask_bias):
  n, d = x.shape
  e = edge_index.shape[1]
  out_dim = W_sd.shape[1]

  n_pad = _ceil_to(n, _NS * _CHUNK)
  e_pad = _ceil_to(e, _NS * _CHUNK * 2)

  x_pad = jnp.pad(x, ((0, n_pad - n), (0, 0)))
  od_pad = jnp.pad(out_degree, (0, n_pad - n)).reshape(n_pad, 1)
  id_pad = jnp.pad(in_degree, (0, n_pad - n)).reshape(n_pad, 1)
  row_p = jnp.pad(edge_index[0], (0, e_pad - e), constant_values=n_pad - 1)
  col_p = jnp.pad(edge_index[1], (0, e_pad - e), constant_values=0)

  y, z = _prologue(x_pad, od_pad, id_pad, n_pad, d)
  if _PACKED:
    y = jax.lax.bitcast_convert_type(y.reshape(n_pad, d // 2, 2), jnp.int32)
    z = jax.lax.bitcast_convert_type(z.reshape(n_pad, d // 2, 2), jnp.int32)

  zeros_tile = jnp.zeros((n_pad // _NS, d), jnp.float32)
  odeg_flat = od_pad.reshape(n_pad)
  ideg_flat = id_pad.reshape(n_pad)
  oacc, iacc, otg, itg = _sc_aggregate(
      y, z, row_p, col_p, odeg_flat, ideg_flat, out_tab, in_tab,
      zeros_tile, n_pad, d, e_pad)

  if _PACKED:
    sigma = np.empty(d, dtype=np.int32)
    for g in range(d // 32):
      for i in range(16):
        sigma[32 * g + i] = 32 * g + 2 * i
        sigma[32 * g + 16 + i] = 32 * g + 2 * i + 1
  else:
    sigma = np.arange(d, dtype=np.int32)

  pad1 = lambda v: jnp.pad(v, (0, n_pad - n)).reshape(n_pad, 1)
  out, co, ci = _epilogue(
      x_pad, oacc, iacc, otg, itg, od_pad, id_pad,
      pad1(out_deg_mask), pad1(out_deg_mask_bias),
      pad1(in_deg_mask), pad1(in_deg_mask_bias),
      W_sd[sigma, :], b_sd.reshape(1, out_dim),
      W_ds[sigma, :], b_ds.reshape(1, out_dim),
      w_out_f.reshape(1, d), w_out_f[sigma, :].reshape(1, d),
      b_out_f.reshape(1, 1),
      w_in_f.reshape(1, d), w_in_f[sigma, :].reshape(1, d),
      b_in_f.reshape(1, 1),
      W_fc, b_fc.reshape(1, out_dim),
      n_pad, d, out_dim)

  return out[:n], ci[:n], co[:n]


def kernel(x, edge_index, in_degree, out_degree, in_tab, out_tab,
           W_sd, b_sd, W_ds, b_ds, w_out_f, b_out_f, w_in_f, b_in_f,
           W_fc, b_fc, out_deg_mask, out_deg_mask_bias,
           in_deg_mask, in_deg_mask_bias):
  return _run(x, edge_index, in_degree, out_degree, in_tab, out_tab,
              W_sd, b_sd, W_ds, b_ds, w_out_f, b_out_f, w_in_f, b_in_f,
              W_fc, b_fc, out_deg_mask, out_deg_mask_bias,
              in_deg_mask, in_deg_mask_bias)
